# Initial kernel scaffold; baseline (speedup 1.0000x reference)
import jax
import jax.numpy as jnp
from jax.experimental import pallas as pl


def kernel(atom_type, pos, current_edge_index, current_edge_feat, full_edge_index, full_edge_type, atom_emb, bond_emb, W_d1, b_d1, W_d2, b_d2, W_e1, b_e1, W_e2, b_e2, W_s1, b_s1, W_s2, b_s2, W_s3, b_s3):
    raise NotImplementedError("write your pallas kernel here")


# speedup vs baseline: 1.0726x; 1.0726x over previous
import functools

import jax
import jax.numpy as jnp
from jax.experimental import pallas as pl
from jax.experimental.pallas import tpu as pltpu

NB = 8

_BE = 4000
_BN = 2000


def _edge_attr_body(d2_ref, et_ref, wd1_ref, bd1_ref, wd2_ref, bd2_ref,
                    bond_ref, out_ref):
    d = jnp.sqrt(d2_ref[...])
    h = jax.nn.relu(d * wd1_ref[...] + bd1_ref[...])
    h = jnp.dot(h, wd2_ref[...], preferred_element_type=jnp.float32) + bd2_ref[...]
    onehot = (et_ref[...] == jax.lax.broadcasted_iota(jnp.int32, (1, NB), 1)
              ).astype(jnp.float32)
    battr = jnp.dot(onehot, bond_ref[...], preferred_element_type=jnp.float32)
    out_ref[...] = h * battr


def _edge_attr(d2, etype, W_d1, b_d1, W_d2, b_d2, bond_emb):
    E = d2.shape[0]
    H = W_d1.shape[1]
    grid = (E // _BE,)
    full = lambda i: (0, 0)
    return pl.pallas_call(
        _edge_attr_body,
        grid=grid,
        in_specs=[
            pl.BlockSpec((_BE, 1), lambda i: (i, 0)),
            pl.BlockSpec((_BE, 1), lambda i: (i, 0)),
            pl.BlockSpec((1, H), full),
            pl.BlockSpec((1, H), full),
            pl.BlockSpec((H, H), full),
            pl.BlockSpec((1, H), full),
            pl.BlockSpec((NB, H), full),
        ],
        out_specs=pl.BlockSpec((_BE, H), lambda i: (i, 0)),
        out_shape=jax.ShapeDtypeStruct((E, H), jnp.float32),
    )(d2, etype, W_d1, b_d1.reshape(1, H), W_d2, b_d2.reshape(1, H), bond_emb)


def _node_body(z_ref, agg_ref, w_ref, b_ref, out_ref):
    x = z_ref[...] + agg_ref[...]
    out_ref[...] = jax.nn.relu(
        jnp.dot(x, w_ref[...], preferred_element_type=jnp.float32) + b_ref[...])


def _node_update(z, agg, W, b):
    Np, H = z.shape
    grid = (Np // _BN,)
    full = lambda i: (0, 0)
    return pl.pallas_call(
        _node_body,
        grid=grid,
        in_specs=[
            pl.BlockSpec((_BN, H), lambda i: (i, 0)),
            pl.BlockSpec((_BN, H), lambda i: (i, 0)),
            pl.BlockSpec((H, H), full),
            pl.BlockSpec((1, H), full),
        ],
        out_specs=pl.BlockSpec((_BN, H), lambda i: (i, 0)),
        out_shape=jax.ShapeDtypeStruct((Np, H), jnp.float32),
    )(z, agg, W, b.reshape(1, H))


def _score_body(pair_ref, e2_ref, wa_ref, wb_ref, b1_ref, w2_ref, b2_ref,
                w3_ref, b3_ref, out_ref):
    x = jax.nn.relu(
        jnp.dot(pair_ref[...], wa_ref[...], preferred_element_type=jnp.float32)
        + jnp.dot(e2_ref[...], wb_ref[...], preferred_element_type=jnp.float32)
        + b1_ref[...])
    x = jax.nn.relu(
        jnp.dot(x, w2_ref[...], preferred_element_type=jnp.float32) + b2_ref[...])
    out_ref[...] = jnp.sum(x * w3_ref[...], axis=1, keepdims=True) + b3_ref[...]


def _score_mlp(pair, e2, W_s1, b_s1, W_s2, b_s2, W_s3, b_s3):
    E, H = pair.shape
    Hh = W_s2.shape[1]
    grid = (E // _BE,)
    full = lambda i: (0, 0)
    return pl.pallas_call(
        _score_body,
        grid=grid,
        in_specs=[
            pl.BlockSpec((_BE, H), lambda i: (i, 0)),
            pl.BlockSpec((_BE, H), lambda i: (i, 0)),
            pl.BlockSpec((H, H), full),
            pl.BlockSpec((H, H), full),
            pl.BlockSpec((1, H), full),
            pl.BlockSpec((H, Hh), full),
            pl.BlockSpec((1, Hh), full),
            pl.BlockSpec((1, Hh), full),
            pl.BlockSpec((1, 1), full),
        ],
        out_specs=pl.BlockSpec((_BE, 1), lambda i: (i, 0)),
        out_shape=jax.ShapeDtypeStruct((E, 1), jnp.float32),
    )(pair, e2, W_s1[:H], W_s1[H:], b_s1.reshape(1, H), W_s2,
      b_s2.reshape(1, Hh), W_s3.reshape(1, Hh), b_s3.reshape(1, 1))


def _pair_d2(pos, edge_index):
    dp = pos[edge_index[0]] - pos[edge_index[1]]
    return jnp.sum(dp * dp, axis=-1, keepdims=True)


def kernel(atom_type, pos, current_edge_index, current_edge_feat,
           full_edge_index, full_edge_type,
           atom_emb, bond_emb,
           W_d1, b_d1, W_d2, b_d2,
           W_e1, b_e1, W_e2, b_e2,
           W_s1, b_s1, W_s2, b_s2, W_s3, b_s3):
    n = atom_type.shape[0]
    e = current_edge_index.shape[1]

    z = atom_emb[atom_type]

    edge_attr = _edge_attr(_pair_d2(pos, current_edge_index),
                           current_edge_feat.reshape(e, 1).astype(jnp.int32),
                           W_d1, b_d1, W_d2, b_d2, bond_emb)
    edge2 = _edge_attr(_pair_d2(pos, full_edge_index),
                       full_edge_type.reshape(e, 1).astype(jnp.int32),
                       W_d1, b_d1, W_d2, b_d2, bond_emb)

    src = current_edge_index[0]
    dst = current_edge_index[1]

    msg = z[src] * edge_attr
    agg = jax.ops.segment_sum(msg, dst, num_segments=n)
    h = _node_update(z, agg, W_e1, b_e1)

    msg = h[src] * edge_attr
    agg = jax.ops.segment_sum(msg, dst, num_segments=n)
    node = _node_update(h, agg, W_e2, b_e2)

    pair = node[full_edge_index[0]] * node[full_edge_index[1]]
    return _score_mlp(pair, edge2, W_s1, b_s1, W_s2, b_s2, W_s3, b_s3)


# speedup vs baseline: 1.6513x; 1.5396x over previous
import functools

import jax
import jax.numpy as jnp
from jax import lax
from jax.experimental import pallas as pl
from jax.experimental.pallas import tpu as pltpu
from jax.experimental.pallas import tpu_sc as plsc

NB = 8

_BE = 4000
_BN = 2000

_NC = 2
_NS = 16
_NW = _NC * _NS
_CH = 80


def _make_msg_agg(n, e, h):
    epw = e // _NW
    nchunks = epw // _CH
    n_pad = (n + 8 * _NS - 1) // (8 * _NS) * (8 * _NS)
    rps = n_pad // _NS
    mesh = plsc.VectorSubcoreMesh(core_axis_name="c", subcore_axis_name="s")

    @functools.partial(
        pl.kernel, mesh=mesh,
        out_type=jax.ShapeDtypeStruct((_NC, n_pad, h), jnp.float32),
        scratch_types=[
            pltpu.VMEM((_CH,), jnp.int32),
            pltpu.VMEM((_CH,), jnp.int32),
            pltpu.VMEM((_CH, h), jnp.float32),
            pltpu.VMEM((_CH, h), jnp.float32),
            pltpu.VMEM_SHARED((n_pad, h), jnp.float32),
            pltpu.SemaphoreType.DMA,
        ],
    )
    def msg_agg(z_hbm, ea_hbm, src_hbm, dst_hbm, zero_hbm, out_hbm,
                sidx, didx, rows, eab, acc, sem):
        c = lax.axis_index("c")
        s = lax.axis_index("s")
        wid = s * _NC + c
        pltpu.sync_copy(zero_hbm, acc.at[pl.ds(s * rps, rps)])
        plsc.subcore_barrier()
        base = wid * epw

        def body(i, carry):
            off = base + i * _CH
            pltpu.sync_copy(src_hbm.at[pl.ds(off, _CH)], sidx)
            pltpu.sync_copy(dst_hbm.at[pl.ds(off, _CH)], didx)
            pltpu.sync_copy(ea_hbm.at[pl.ds(off, _CH), :], eab)
            pltpu.async_copy(z_hbm.at[sidx], rows, sem).wait()

            def mrow(r, carry2):
                for k in range(h // 16):
                    sl = pl.ds(k * 16, 16)
                    rows[r, sl] = rows[r, sl] * eab[r, sl]
                return carry2

            lax.fori_loop(0, _CH, mrow, 0)
            pltpu.sync_copy(rows, acc.at[didx], add=True)
            return carry

        lax.fori_loop(0, nchunks, body, 0)
        plsc.subcore_barrier()
        sl = pl.ds(s * rps, rps)
        pltpu.sync_copy(acc.at[sl], out_hbm.at[c, sl, :])

    return msg_agg


def _make_pair_prod(n, e, h):
    epw = e // _NW
    nchunks = epw // _CH
    mesh = plsc.VectorSubcoreMesh(core_axis_name="c", subcore_axis_name="s")

    @functools.partial(
        pl.kernel, mesh=mesh,
        out_type=jax.ShapeDtypeStruct((e, h), jnp.float32),
        scratch_types=[
            pltpu.VMEM((_CH,), jnp.int32),
            pltpu.VMEM((_CH,), jnp.int32),
            pltpu.VMEM((_CH, h), jnp.float32),
            pltpu.VMEM((_CH, h), jnp.float32),
            pltpu.SemaphoreType.DMA,
        ],
    )
    def pair_prod(node_hbm, src_hbm, dst_hbm, out_hbm,
                  sidx, didx, rows, rows2, sem):
        c = lax.axis_index("c")
        s = lax.axis_index("s")
        wid = s * _NC + c
        base = wid * epw

        def body(i, carry):
            off = base + i * _CH
            pltpu.sync_copy(src_hbm.at[pl.ds(off, _CH)], sidx)
            pltpu.sync_copy(dst_hbm.at[pl.ds(off, _CH)], didx)
            pltpu.async_copy(node_hbm.at[sidx], rows, sem).wait()
            pltpu.async_copy(node_hbm.at[didx], rows2, sem).wait()

            def mrow(r, carry2):
                for k in range(h // 16):
                    sl = pl.ds(k * 16, 16)
                    rows[r, sl] = rows[r, sl] * rows2[r, sl]
                return carry2

            lax.fori_loop(0, _CH, mrow, 0)
            pltpu.sync_copy(rows, out_hbm.at[pl.ds(off, _CH), :])
            return carry

        lax.fori_loop(0, nchunks, body, 0)

    return pair_prod


def _edge_attr_body(d2_ref, et_ref, wd1_ref, bd1_ref, wd2_ref, bd2_ref,
                    bond_ref, out_ref):
    d = jnp.sqrt(d2_ref[...])
    h = jax.nn.relu(d * wd1_ref[...] + bd1_ref[...])
    h = jnp.dot(h, wd2_ref[...], preferred_element_type=jnp.float32) + bd2_ref[...]
    onehot = (et_ref[...] == jax.lax.broadcasted_iota(jnp.int32, (1, NB), 1)
              ).astype(jnp.float32)
    battr = jnp.dot(onehot, bond_ref[...], preferred_element_type=jnp.float32)
    out_ref[...] = h * battr


def _edge_attr(d2, etype, W_d1, b_d1, W_d2, b_d2, bond_emb):
    E = d2.shape[0]
    H = W_d1.shape[1]
    grid = (E // _BE,)
    full = lambda i: (0, 0)
    return pl.pallas_call(
        _edge_attr_body,
        grid=grid,
        in_specs=[
            pl.BlockSpec((_BE, 1), lambda i: (i, 0)),
            pl.BlockSpec((_BE, 1), lambda i: (i, 0)),
            pl.BlockSpec((1, H), full),
            pl.BlockSpec((1, H), full),
            pl.BlockSpec((H, H), full),
            pl.BlockSpec((1, H), full),
            pl.BlockSpec((NB, H), full),
        ],
        out_specs=pl.BlockSpec((_BE, H), lambda i: (i, 0)),
        out_shape=jax.ShapeDtypeStruct((E, H), jnp.float32),
    )(d2, etype, W_d1, b_d1.reshape(1, H), W_d2, b_d2.reshape(1, H), bond_emb)


def _node_body(z_ref, agg_ref, w_ref, b_ref, out_ref):
    x = z_ref[...] + agg_ref[0] + agg_ref[1]
    out_ref[...] = jax.nn.relu(
        jnp.dot(x, w_ref[...], preferred_element_type=jnp.float32) + b_ref[...])


def _node_update(z, agg2, W, b):
    Np, H = z.shape
    grid = (Np // _BN,)
    full = lambda i: (0, 0)
    return pl.pallas_call(
        _node_body,
        grid=grid,
        in_specs=[
            pl.BlockSpec((_BN, H), lambda i: (i, 0)),
            pl.BlockSpec((2, _BN, H), lambda i: (0, i, 0)),
            pl.BlockSpec((H, H), full),
            pl.BlockSpec((1, H), full),
        ],
        out_specs=pl.BlockSpec((_BN, H), lambda i: (i, 0)),
        out_shape=jax.ShapeDtypeStruct((Np, H), jnp.float32),
    )(z, agg2, W, b.reshape(1, H))


def _score_body(pair_ref, e2_ref, wa_ref, wb_ref, b1_ref, w2_ref, b2_ref,
                w3_ref, b3_ref, out_ref):
    x = jax.nn.relu(
        jnp.dot(pair_ref[...], wa_ref[...], preferred_element_type=jnp.float32)
        + jnp.dot(e2_ref[...], wb_ref[...], preferred_element_type=jnp.float32)
        + b1_ref[...])
    x = jax.nn.relu(
        jnp.dot(x, w2_ref[...], preferred_element_type=jnp.float32) + b2_ref[...])
    out_ref[...] = jnp.sum(x * w3_ref[...], axis=1, keepdims=True) + b3_ref[...]


def _score_mlp(pair, e2, W_s1, b_s1, W_s2, b_s2, W_s3, b_s3):
    E, H = pair.shape
    Hh = W_s2.shape[1]
    grid = (E // _BE,)
    full = lambda i: (0, 0)
    return pl.pallas_call(
        _score_body,
        grid=grid,
        in_specs=[
            pl.BlockSpec((_BE, H), lambda i: (i, 0)),
            pl.BlockSpec((_BE, H), lambda i: (i, 0)),
            pl.BlockSpec((H, H), full),
            pl.BlockSpec((H, H), full),
            pl.BlockSpec((1, H), full),
            pl.BlockSpec((H, Hh), full),
            pl.BlockSpec((1, Hh), full),
            pl.BlockSpec((1, Hh), full),
            pl.BlockSpec((1, 1), full),
        ],
        out_specs=pl.BlockSpec((_BE, 1), lambda i: (i, 0)),
        out_shape=jax.ShapeDtypeStruct((E, 1), jnp.float32),
    )(pair, e2, W_s1[:H], W_s1[H:], b_s1.reshape(1, H), W_s2,
      b_s2.reshape(1, Hh), W_s3.reshape(1, Hh), b_s3.reshape(1, 1))


def _pair_d2(pos, edge_index):
    dp = pos[edge_index[0]] - pos[edge_index[1]]
    return jnp.sum(dp * dp, axis=-1, keepdims=True)


def kernel(atom_type, pos, current_edge_index, current_edge_feat,
           full_edge_index, full_edge_type,
           atom_emb, bond_emb,
           W_d1, b_d1, W_d2, b_d2,
           W_e1, b_e1, W_e2, b_e2,
           W_s1, b_s1, W_s2, b_s2, W_s3, b_s3):
    n = atom_type.shape[0]
    e = current_edge_index.shape[1]

    z = atom_emb[atom_type]

    edge_attr = _edge_attr(_pair_d2(pos, current_edge_index),
                           current_edge_feat.reshape(e, 1).astype(jnp.int32),
                           W_d1, b_d1, W_d2, b_d2, bond_emb)
    edge2 = _edge_attr(_pair_d2(pos, full_edge_index),
                       full_edge_type.reshape(e, 1).astype(jnp.int32),
                       W_d1, b_d1, W_d2, b_d2, bond_emb)

    src = current_edge_index[0]
    dst = current_edge_index[1]
    n_pad = (n + 8 * _NS - 1) // (8 * _NS) * (8 * _NS)
    zeros = jnp.zeros((n_pad // _NS, z.shape[1]), jnp.float32)

    msg_agg = _make_msg_agg(n, e, z.shape[1])
    agg2 = msg_agg(z, edge_attr, src, dst, zeros)
    h = _node_update(z, agg2, W_e1, b_e1)

    agg2 = msg_agg(h, edge_attr, src, dst, zeros)
    node = _node_update(h, agg2, W_e2, b_e2)

    pair = _make_pair_prod(n, e, z.shape[1])(
        node, full_edge_index[0], full_edge_index[1])
    return _score_mlp(pair, edge2, W_s1, b_s1, W_s2, b_s2, W_s3, b_s3)


# speedup vs baseline: 4.7252x; 2.8614x over previous
import functools

import jax
import jax.numpy as jnp
from jax import lax
from jax.experimental import pallas as pl
from jax.experimental.pallas import tpu as pltpu
from jax.experimental.pallas import tpu_sc as plsc

NB = 8

_BE = 4000
_BN = 2000

_NC = 2
_NS = 16
_NW = _NC * _NS
_CH = 80


_K = 4


def _mul_rows(dst_ref, a_ref, b_ref, nrows, h):
    def mrow(r, carry):
        for k in range(h // 16):
            sl = pl.ds(k * 16, 16)
            dst_ref[r, sl] = a_ref[r, sl] * b_ref[r, sl]
        return carry

    lax.fori_loop(0, nrows, mrow, 0)


def _make_msg_agg(n, e, h):
    epw = e // _NW
    nchunks = epw // _CH
    ngroups = (nchunks + 3) // 4
    n_pad = (n + 8 * _NS - 1) // (8 * _NS) * (8 * _NS)
    rps = n_pad // _NS
    mesh = plsc.VectorSubcoreMesh(core_axis_name="c", subcore_axis_name="s")

    @functools.partial(
        pl.kernel, mesh=mesh,
        out_type=jax.ShapeDtypeStruct((_NC, n_pad, h), jnp.float32),
        scratch_types=(
            [pltpu.VMEM((_CH,), jnp.int32) for _ in range(8)]
            + [pltpu.VMEM((_CH, h), jnp.float32) for _ in range(4)]
            + [pltpu.VMEM_SHARED((n_pad, h), jnp.float32)]
            + [pltpu.SemaphoreType.DMA for _ in range(10)]
        ),
    )
    def msg_agg(z_hbm, ea_hbm, src_hbm, dst_hbm, zero_hbm, out_hbm, *bufs):
        sidx = bufs[0:4]
        didx = bufs[4:8]
        rows = bufs[8:10]
        eab = bufs[10:12]
        acc = bufs[12]
        sem_g = bufs[13:15]
        sem_e = bufs[15:17]
        sem_s = bufs[17:19]
        sem_i = bufs[19:23]
        c = lax.axis_index("c")
        s = lax.axis_index("s")
        wid = s * _NC + c
        base = wid * epw
        pltpu.sync_copy(zero_hbm, acc.at[pl.ds(s * rps, rps)])
        plsc.subcore_barrier()

        def issue_idx(i, q):
            off = base + i * _CH
            pltpu.async_copy(src_hbm.at[pl.ds(off, _CH)], sidx[q], sem_i[q])
            pltpu.async_copy(dst_hbm.at[pl.ds(off, _CH)], didx[q], sem_i[q])

        def wait_idx(q):
            pltpu.make_async_copy(
                src_hbm.at[pl.ds(0, _CH)], sidx[q], sem_i[q]).wait()
            pltpu.make_async_copy(
                dst_hbm.at[pl.ds(0, _CH)], didx[q], sem_i[q]).wait()

        def issue_in(i, b, q):
            pltpu.async_copy(
                ea_hbm.at[pl.ds(base + i * _CH, _CH), :], eab[b], sem_e[b])
            pltpu.async_copy(z_hbm.at[sidx[q]], rows[b], sem_g[b])

        issue_idx(0, 0)
        issue_idx(1, 1)
        wait_idx(0)
        issue_in(0, 0, 0)

        def group(g, carry):
            for bb in range(4):
                i = g * 4 + bb
                b = bb % 2
                q = bb
                b1 = (bb + 1) % 2
                q1 = (bb + 1) % 4
                q2 = (bb + 2) % 4

                @pl.when(i < nchunks)
                def _main():
                    pltpu.make_async_copy(
                        z_hbm.at[pl.ds(0, _CH), :], rows[b], sem_g[b]).wait()
                    pltpu.make_async_copy(
                        ea_hbm.at[pl.ds(0, _CH), :], eab[b], sem_e[b]).wait()
                    _mul_rows(rows[b], rows[b], eab[b], _CH, h)
                    pltpu.async_copy(rows[b], acc.at[didx[q]], sem_s[b],
                                     add=True)

                    @pl.when(i + 1 < nchunks)
                    def _prefetch():
                        @pl.when(i >= 1)
                        def _wait_prev_scatter():
                            pltpu.make_async_copy(
                                z_hbm.at[pl.ds(0, _CH), :], rows[b1],
                                sem_s[b1]).wait()

                        wait_idx(q1)
                        issue_in(i + 1, b1, q1)

                    @pl.when(i + 2 < nchunks)
                    def _prefetch_idx():
                        issue_idx(i + 2, q2)

            return carry

        lax.fori_loop(0, ngroups, group, 0)
        pltpu.make_async_copy(
            z_hbm.at[pl.ds(0, _CH), :], rows[(nchunks - 1) % 2],
            sem_s[(nchunks - 1) % 2]).wait()
        pltpu.make_async_copy(
            z_hbm.at[pl.ds(0, _CH), :], rows[nchunks % 2],
            sem_s[nchunks % 2]).wait()
        plsc.subcore_barrier()
        sl = pl.ds(s * rps, rps)
        pltpu.sync_copy(acc.at[sl], out_hbm.at[c, sl, :])

    return msg_agg


def _make_pair_prod(n, e, h):
    epw = e // _NW
    nchunks = epw // _CH
    nmain = (nchunks - 1) // _K * _K
    mesh = plsc.VectorSubcoreMesh(core_axis_name="c", subcore_axis_name="s")

    @functools.partial(
        pl.kernel, mesh=mesh,
        out_type=jax.ShapeDtypeStruct((e, h), jnp.float32),
        scratch_types=(
            [pltpu.VMEM((nchunks, _CH), jnp.int32),
             pltpu.VMEM((nchunks, _CH), jnp.int32)]
            + [pltpu.VMEM((_CH, h), jnp.float32) for _ in range(2 * _K)]
            + [pltpu.SemaphoreType.DMA for _ in range(3 * _K)]
        ),
    )
    def pair_prod(node_hbm, src_hbm, dst_hbm, out_hbm, sidx2, didx2, *bufs):
        rows = bufs[0:_K]
        rows2 = bufs[_K:2 * _K]
        sem_g = bufs[2 * _K:3 * _K]
        sem_g2 = bufs[3 * _K:4 * _K]
        sem_w = bufs[4 * _K:5 * _K]
        c = lax.axis_index("c")
        s = lax.axis_index("s")
        wid = s * _NC + c
        base = wid * epw
        pltpu.sync_copy(src_hbm.at[wid], sidx2)
        pltpu.sync_copy(dst_hbm.at[wid], didx2)

        def issue_in(i, b):
            pltpu.async_copy(node_hbm.at[sidx2.at[i]], rows[b], sem_g[b])
            pltpu.async_copy(node_hbm.at[didx2.at[i]], rows2[b], sem_g2[b])

        for b in range(2):
            issue_in(b, b)

        def group(g, carry):
            for b in range(_K):
                i = g * _K + b
                j = i + 2
                bj = (b + 2) % _K

                @pl.when(jnp.logical_and(j < nmain, j >= _K))
                def _wait_prev():
                    pltpu.make_async_copy(
                        node_hbm.at[pl.ds(0, _CH), :], rows[bj], sem_w[bj]
                    ).wait()

                @pl.when(j < nmain)
                def _prefetch():
                    issue_in(j, bj)

                pltpu.make_async_copy(
                    node_hbm.at[pl.ds(0, _CH), :], rows[b], sem_g[b]).wait()
                pltpu.make_async_copy(
                    node_hbm.at[pl.ds(0, _CH), :], rows2[b], sem_g2[b]).wait()
                _mul_rows(rows[b], rows[b], rows2[b], _CH, h)
                pltpu.async_copy(
                    rows[b], out_hbm.at[pl.ds(base + i * _CH, _CH), :],
                    sem_w[b])
            return carry

        lax.fori_loop(0, nmain // _K, group, 0)
        for b in range(_K):
            pltpu.make_async_copy(
                node_hbm.at[pl.ds(0, _CH), :], rows[b], sem_w[b]).wait()
        for i in range(nmain, nchunks):
            pltpu.async_copy(node_hbm.at[sidx2.at[i]], rows[0],
                             sem_g[0]).wait()
            pltpu.async_copy(node_hbm.at[didx2.at[i]], rows2[0],
                             sem_g2[0]).wait()
            _mul_rows(rows[0], rows[0], rows2[0], _CH, h)
            pltpu.async_copy(
                rows[0], out_hbm.at[pl.ds(base + i * _CH, _CH), :],
                sem_w[0]).wait()

    return pair_prod


def _make_edge_d2(n, e):
    epw = e // _NW
    nchunks = epw // _CH
    prows = (3 * n + 127) // 128
    mesh = plsc.VectorSubcoreMesh(core_axis_name="c", subcore_axis_name="s")

    @functools.partial(
        pl.kernel, mesh=mesh,
        compiler_params=pltpu.CompilerParams(needs_layout_passes=False),
        out_type=(jax.ShapeDtypeStruct((e,), jnp.float32),
                  jax.ShapeDtypeStruct((e,), jnp.float32)),
        scratch_types=[
            pltpu.VMEM((prows, 128), jnp.float32),
            pltpu.VMEM((epw,), jnp.int32),
            pltpu.VMEM((epw,), jnp.int32),
            pltpu.VMEM((epw,), jnp.float32),
        ],
    )
    def edge_d2(pos_hbm, srcc_hbm, dstc_hbm, srcf_hbm, dstf_hbm,
                outc_hbm, outf_hbm, posv, sidx, didx, d2v):
        c = lax.axis_index("c")
        s = lax.axis_index("s")
        wid = s * _NC + c
        pltpu.sync_copy(pos_hbm, posv)
        base = wid * epw

        for src_hbm, dst_hbm, out_hbm in ((srcc_hbm, dstc_hbm, outc_hbm),
                                          (srcf_hbm, dstf_hbm, outf_hbm)):
            pltpu.sync_copy(src_hbm.at[pl.ds(base, epw)], sidx)
            pltpu.sync_copy(dst_hbm.at[pl.ds(base, epw)], didx)

            def veci(k, carry2):
                sl = pl.ds(k * 16, 16)
                si = sidx[sl] * 3
                di = didx[sl] * 3
                acc = jnp.zeros((16,), jnp.float32)
                for j in range(3):
                    sij = si + j
                    dij = di + j
                    dp = (plsc.load_gather(posv, [sij >> 7, sij & 127])
                          - plsc.load_gather(posv, [dij >> 7, dij & 127]))
                    acc = acc + dp * dp
                d2v[sl] = acc
                return carry2

            lax.fori_loop(0, epw // 16, veci, 0)
            pltpu.sync_copy(d2v, out_hbm.at[pl.ds(base, epw)])

    return edge_d2


def _edge_attr_body(d2_ref, et_ref, wd1_ref, bd1_ref, wd2_ref, bd2_ref,
                    bond_ref, out_ref):
    d = jnp.sqrt(d2_ref[...])
    h = jax.nn.relu(d * wd1_ref[...] + bd1_ref[...])
    h = jnp.dot(h, wd2_ref[...], preferred_element_type=jnp.float32) + bd2_ref[...]
    onehot = (et_ref[...] == jax.lax.broadcasted_iota(jnp.int32, (1, NB), 1)
              ).astype(jnp.float32)
    battr = jnp.dot(onehot, bond_ref[...], preferred_element_type=jnp.float32)
    out_ref[...] = h * battr


def _edge_attr(d2, etype, W_d1, b_d1, W_d2, b_d2, bond_emb):
    E = d2.shape[0]
    H = W_d1.shape[1]
    grid = (E // _BE,)
    full = lambda i: (0, 0)
    return pl.pallas_call(
        _edge_attr_body,
        grid=grid,
        in_specs=[
            pl.BlockSpec((_BE, 1), lambda i: (i, 0)),
            pl.BlockSpec((_BE, 1), lambda i: (i, 0)),
            pl.BlockSpec((1, H), full),
            pl.BlockSpec((1, H), full),
            pl.BlockSpec((H, H), full),
            pl.BlockSpec((1, H), full),
            pl.BlockSpec((NB, H), full),
        ],
        out_specs=pl.BlockSpec((_BE, H), lambda i: (i, 0)),
        out_shape=jax.ShapeDtypeStruct((E, H), jnp.float32),
    )(d2, etype, W_d1, b_d1.reshape(1, H), W_d2, b_d2.reshape(1, H), bond_emb)


def _node_body(z_ref, agg_ref, w_ref, b_ref, out_ref):
    x = z_ref[...] + agg_ref[0] + agg_ref[1]
    out_ref[...] = jax.nn.relu(
        jnp.dot(x, w_ref[...], preferred_element_type=jnp.float32) + b_ref[...])


def _node_update(z, agg2, W, b):
    Np, H = z.shape
    grid = (Np // _BN,)
    full = lambda i: (0, 0)
    return pl.pallas_call(
        _node_body,
        grid=grid,
        in_specs=[
            pl.BlockSpec((_BN, H), lambda i: (i, 0)),
            pl.BlockSpec((2, _BN, H), lambda i: (0, i, 0)),
            pl.BlockSpec((H, H), full),
            pl.BlockSpec((1, H), full),
        ],
        out_specs=pl.BlockSpec((_BN, H), lambda i: (i, 0)),
        out_shape=jax.ShapeDtypeStruct((Np, H), jnp.float32),
    )(z, agg2, W, b.reshape(1, H))


def _score_body(pair_ref, e2_ref, wa_ref, wb_ref, b1_ref, w2_ref, b2_ref,
                w3_ref, b3_ref, out_ref):
    x = jax.nn.relu(
        jnp.dot(pair_ref[...], wa_ref[...], preferred_element_type=jnp.float32)
        + jnp.dot(e2_ref[...], wb_ref[...], preferred_element_type=jnp.float32)
        + b1_ref[...])
    x = jax.nn.relu(
        jnp.dot(x, w2_ref[...], preferred_element_type=jnp.float32) + b2_ref[...])
    out_ref[...] = jnp.sum(x * w3_ref[...], axis=1, keepdims=True) + b3_ref[...]


def _score_mlp(pair, e2, W_s1, b_s1, W_s2, b_s2, W_s3, b_s3):
    E, H = pair.shape
    Hh = W_s2.shape[1]
    grid = (E // _BE,)
    full = lambda i: (0, 0)
    return pl.pallas_call(
        _score_body,
        grid=grid,
        in_specs=[
            pl.BlockSpec((_BE, H), lambda i: (i, 0)),
            pl.BlockSpec((_BE, H), lambda i: (i, 0)),
            pl.BlockSpec((H, H), full),
            pl.BlockSpec((H, H), full),
            pl.BlockSpec((1, H), full),
            pl.BlockSpec((H, Hh), full),
            pl.BlockSpec((1, Hh), full),
            pl.BlockSpec((1, Hh), full),
            pl.BlockSpec((1, 1), full),
        ],
        out_specs=pl.BlockSpec((_BE, 1), lambda i: (i, 0)),
        out_shape=jax.ShapeDtypeStruct((E, 1), jnp.float32),
    )(pair, e2, W_s1[:H], W_s1[H:], b_s1.reshape(1, H), W_s2,
      b_s2.reshape(1, Hh), W_s3.reshape(1, Hh), b_s3.reshape(1, 1))


def kernel(atom_type, pos, current_edge_index, current_edge_feat,
           full_edge_index, full_edge_type,
           atom_emb, bond_emb,
           W_d1, b_d1, W_d2, b_d2,
           W_e1, b_e1, W_e2, b_e2,
           W_s1, b_s1, W_s2, b_s2, W_s3, b_s3):
    n = atom_type.shape[0]
    e = current_edge_index.shape[1]

    z = atom_emb[atom_type]

    prows = (3 * n + 127) // 128
    pos_pad = jnp.zeros((prows * 128,), jnp.float32).at[:3 * n].set(
        pos.reshape(-1)).reshape(prows, 128)
    d2c, d2f = _make_edge_d2(n, e)(
        pos_pad, current_edge_index[0], current_edge_index[1],
        full_edge_index[0], full_edge_index[1])
    edge_attr = _edge_attr(d2c.reshape(e, 1),
                           current_edge_feat.reshape(e, 1).astype(jnp.int32),
                           W_d1, b_d1, W_d2, b_d2, bond_emb)
    edge2 = _edge_attr(d2f.reshape(e, 1),
                       full_edge_type.reshape(e, 1).astype(jnp.int32),
                       W_d1, b_d1, W_d2, b_d2, bond_emb)

    nchunks = e // (_NW * _CH)
    fsrc3 = full_edge_index[0].reshape(_NW, nchunks, _CH)
    fdst3 = full_edge_index[1].reshape(_NW, nchunks, _CH)
    n_pad = (n + 8 * _NS - 1) // (8 * _NS) * (8 * _NS)
    zeros = jnp.zeros((n_pad // _NS, z.shape[1]), jnp.float32)

    msg_agg = _make_msg_agg(n, e, z.shape[1])
    agg2 = msg_agg(z, edge_attr, current_edge_index[0],
                   current_edge_index[1], zeros)
    h = _node_update(z, agg2, W_e1, b_e1)

    agg2 = msg_agg(h, edge_attr, current_edge_index[0],
                   current_edge_index[1], zeros)
    node = _node_update(h, agg2, W_e2, b_e2)

    pair = _make_pair_prod(n, e, z.shape[1])(node, fsrc3, fdst3)
    return _score_mlp(pair, edge2, W_s1, b_s1, W_s2, b_s2, W_s3, b_s3)


# speedup vs baseline: 6.7283x; 1.4239x over previous
import functools

import jax
import jax.numpy as jnp
from jax import lax
from jax.experimental import pallas as pl
from jax.experimental.pallas import tpu as pltpu
from jax.experimental.pallas import tpu_sc as plsc

NB = 8

_BE = 4000
_BN = 2000

_NC = 2
_NS = 16
_NW = _NC * _NS
_CH = 80


_K = 4


def _mul_rows(dst_ref, a_ref, b_ref, nrows, h):
    def mrow(r, carry):
        for k in range(h // 16):
            sl = pl.ds(k * 16, 16)
            dst_ref[r, sl] = a_ref[r, sl] * b_ref[r, sl]
        return carry

    lax.fori_loop(0, nrows, mrow, 0)


def _make_msg_agg(n, e, h):
    epw = e // _NW
    nchunks = epw // _CH
    ngroups = (nchunks + 3) // 4
    n_pad = (n + 8 * _NS - 1) // (8 * _NS) * (8 * _NS)
    rps = n_pad // _NS
    mesh = plsc.VectorSubcoreMesh(core_axis_name="c", subcore_axis_name="s")

    @functools.partial(
        pl.kernel, mesh=mesh,
        out_type=jax.ShapeDtypeStruct((_NC, n_pad, h), jnp.float32),
        scratch_types=(
            [pltpu.VMEM((_CH,), jnp.int32) for _ in range(8)]
            + [pltpu.VMEM((_CH, h), jnp.float32) for _ in range(4)]
            + [pltpu.VMEM_SHARED((n_pad, h), jnp.float32)]
            + [pltpu.SemaphoreType.DMA for _ in range(10)]
        ),
    )
    def msg_agg(z_hbm, ea_hbm, src_hbm, dst_hbm, zero_hbm, out_hbm, *bufs):
        sidx = bufs[0:4]
        didx = bufs[4:8]
        rows = bufs[8:10]
        eab = bufs[10:12]
        acc = bufs[12]
        sem_g = bufs[13:15]
        sem_e = bufs[15:17]
        sem_s = bufs[17:19]
        sem_i = bufs[19:23]
        c = lax.axis_index("c")
        s = lax.axis_index("s")
        wid = s * _NC + c
        base = wid * epw
        pltpu.sync_copy(zero_hbm, acc.at[pl.ds(s * rps, rps)])
        plsc.subcore_barrier()

        def issue_idx(i, q):
            off = base + i * _CH
            pltpu.async_copy(src_hbm.at[pl.ds(off, _CH)], sidx[q], sem_i[q])
            pltpu.async_copy(dst_hbm.at[pl.ds(off, _CH)], didx[q], sem_i[q])

        def wait_idx(q):
            pltpu.make_async_copy(
                src_hbm.at[pl.ds(0, _CH)], sidx[q], sem_i[q]).wait()
            pltpu.make_async_copy(
                dst_hbm.at[pl.ds(0, _CH)], didx[q], sem_i[q]).wait()

        def issue_gather(i, b, q):
            pltpu.async_copy(z_hbm.at[sidx[q]], rows[b], sem_g[b])

        def issue_ea(i, b):
            pltpu.async_copy(
                ea_hbm.at[pl.ds(base + i * _CH, _CH), :], eab[b], sem_e[b])

        issue_idx(0, 0)
        issue_idx(1, 1)
        issue_idx(2, 2)
        wait_idx(0)
        issue_gather(0, 0, 0)
        issue_ea(0, 0)
        wait_idx(1)
        issue_gather(1, 1, 1)

        def group(g, carry):
            for bb in range(4):
                i = g * 4 + bb
                b = bb % 2
                q = bb
                b1 = (bb + 1) % 2
                q2 = (bb + 2) % 4
                q3 = (bb + 3) % 4

                @pl.when(i < nchunks)
                def _main():
                    pltpu.make_async_copy(
                        z_hbm.at[pl.ds(0, _CH), :], rows[b], sem_g[b]).wait()
                    pltpu.make_async_copy(
                        ea_hbm.at[pl.ds(0, _CH), :], eab[b], sem_e[b]).wait()
                    _mul_rows(eab[b], rows[b], eab[b], _CH, h)
                    pltpu.async_copy(eab[b], acc.at[didx[q]], sem_s[b],
                                     add=True)

                    @pl.when(i >= 1)
                    def _drain_prev_scatter():
                        pltpu.make_async_copy(
                            z_hbm.at[pl.ds(0, _CH), :], eab[b1],
                            sem_s[b1]).wait()

                    @pl.when(i + 1 < nchunks)
                    def _prefetch_ea():
                        issue_ea(i + 1, b1)

                    @pl.when(i + 3 < nchunks)
                    def _prefetch_idx():
                        issue_idx(i + 3, q3)

                    @pl.when(i + 2 < nchunks)
                    def _prefetch_gather():
                        wait_idx(q2)
                        issue_gather(i + 2, b, q2)

            return carry

        lax.fori_loop(0, ngroups, group, 0)
        pltpu.make_async_copy(
            z_hbm.at[pl.ds(0, _CH), :], eab[(nchunks - 1) % 2],
            sem_s[(nchunks - 1) % 2]).wait()
        plsc.subcore_barrier()
        sl = pl.ds(s * rps, rps)
        pltpu.sync_copy(acc.at[sl], out_hbm.at[c, sl, :])

    return msg_agg


def _make_pair_prod(n, e, h):
    epw = e // _NW
    nchunks = epw // _CH
    nmain = (nchunks - 1) // _K * _K
    mesh = plsc.VectorSubcoreMesh(core_axis_name="c", subcore_axis_name="s")

    @functools.partial(
        pl.kernel, mesh=mesh,
        out_type=jax.ShapeDtypeStruct((e, h), jnp.float32),
        scratch_types=(
            [pltpu.VMEM((nchunks, _CH), jnp.int32),
             pltpu.VMEM((nchunks, _CH), jnp.int32)]
            + [pltpu.VMEM((_CH, h), jnp.float32) for _ in range(2 * _K)]
            + [pltpu.SemaphoreType.DMA for _ in range(3 * _K)]
        ),
    )
    def pair_prod(node_hbm, src_hbm, dst_hbm, out_hbm, sidx2, didx2, *bufs):
        rows = bufs[0:_K]
        rows2 = bufs[_K:2 * _K]
        sem_g = bufs[2 * _K:3 * _K]
        sem_g2 = bufs[3 * _K:4 * _K]
        sem_w = bufs[4 * _K:5 * _K]
        c = lax.axis_index("c")
        s = lax.axis_index("s")
        wid = s * _NC + c
        base = wid * epw
        pltpu.sync_copy(src_hbm.at[wid], sidx2)
        pltpu.sync_copy(dst_hbm.at[wid], didx2)

        def issue_in(i, b):
            pltpu.async_copy(node_hbm.at[sidx2.at[i]], rows[b], sem_g[b])
            pltpu.async_copy(node_hbm.at[didx2.at[i]], rows2[b], sem_g2[b])

        for b in range(2):
            issue_in(b, b)

        def group(g, carry):
            for b in range(_K):
                i = g * _K + b
                j = i + 2
                bj = (b + 2) % _K

                @pl.when(jnp.logical_and(j < nmain, j >= _K))
                def _wait_prev():
                    pltpu.make_async_copy(
                        node_hbm.at[pl.ds(0, _CH), :], rows[bj], sem_w[bj]
                    ).wait()

                @pl.when(j < nmain)
                def _prefetch():
                    issue_in(j, bj)

                pltpu.make_async_copy(
                    node_hbm.at[pl.ds(0, _CH), :], rows[b], sem_g[b]).wait()
                pltpu.make_async_copy(
                    node_hbm.at[pl.ds(0, _CH), :], rows2[b], sem_g2[b]).wait()
                _mul_rows(rows[b], rows[b], rows2[b], _CH, h)
                pltpu.async_copy(
                    rows[b], out_hbm.at[pl.ds(base + i * _CH, _CH), :],
                    sem_w[b])
            return carry

        lax.fori_loop(0, nmain // _K, group, 0)
        for b in range(_K):
            pltpu.make_async_copy(
                node_hbm.at[pl.ds(0, _CH), :], rows[b], sem_w[b]).wait()
        for i in range(nmain, nchunks):
            pltpu.async_copy(node_hbm.at[sidx2.at[i]], rows[0],
                             sem_g[0]).wait()
            pltpu.async_copy(node_hbm.at[didx2.at[i]], rows2[0],
                             sem_g2[0]).wait()
            _mul_rows(rows[0], rows[0], rows2[0], _CH, h)
            pltpu.async_copy(
                rows[0], out_hbm.at[pl.ds(base + i * _CH, _CH), :],
                sem_w[0]).wait()

    return pair_prod


def _make_edge_d2(n, e):
    epw = e // _NW
    nchunks = epw // _CH
    prows = (3 * n + 127) // 128
    mesh = plsc.VectorSubcoreMesh(core_axis_name="c", subcore_axis_name="s")

    @functools.partial(
        pl.kernel, mesh=mesh,
        compiler_params=pltpu.CompilerParams(needs_layout_passes=False),
        out_type=(jax.ShapeDtypeStruct((e,), jnp.float32),
                  jax.ShapeDtypeStruct((e,), jnp.float32)),
        scratch_types=[
            pltpu.VMEM((prows, 128), jnp.float32),
            pltpu.VMEM((epw,), jnp.int32),
            pltpu.VMEM((epw,), jnp.int32),
            pltpu.VMEM((epw,), jnp.float32),
        ],
    )
    def edge_d2(pos_hbm, srcc_hbm, dstc_hbm, srcf_hbm, dstf_hbm,
                outc_hbm, outf_hbm, posv, sidx, didx, d2v):
        c = lax.axis_index("c")
        s = lax.axis_index("s")
        wid = s * _NC + c
        pltpu.sync_copy(pos_hbm, posv)
        base = wid * epw

        for src_hbm, dst_hbm, out_hbm in ((srcc_hbm, dstc_hbm, outc_hbm),
                                          (srcf_hbm, dstf_hbm, outf_hbm)):
            pltpu.sync_copy(src_hbm.at[pl.ds(base, epw)], sidx)
            pltpu.sync_copy(dst_hbm.at[pl.ds(base, epw)], didx)

            def veci(k, carry2):
                sl = pl.ds(k * 16, 16)
                si = sidx[sl] * 3
                di = didx[sl] * 3
                acc = jnp.zeros((16,), jnp.float32)
                for j in range(3):
                    sij = si + j
                    dij = di + j
                    dp = (plsc.load_gather(posv, [sij >> 7, sij & 127])
                          - plsc.load_gather(posv, [dij >> 7, dij & 127]))
                    acc = acc + dp * dp
                d2v[sl] = acc
                return carry2

            lax.fori_loop(0, epw // 16, veci, 0)
            pltpu.sync_copy(d2v, out_hbm.at[pl.ds(base, epw)])

    return edge_d2


_BEA = 2560


def _edge_attr_body(d2_ref, et_ref, wd1_ref, bd1_ref, wd2_ref, bd2_ref,
                    bond_ref, out_ref):
    d = jnp.sqrt(d2_ref[...]).reshape(_BEA, 1)
    h = jax.nn.relu(d * wd1_ref[...] + bd1_ref[...])
    h = jnp.dot(h, wd2_ref[...], preferred_element_type=jnp.float32) + bd2_ref[...]
    et = et_ref[...].reshape(_BEA, 1)
    onehot = (et == jax.lax.broadcasted_iota(jnp.int32, (1, NB), 1)
              ).astype(jnp.float32)
    battr = jnp.dot(onehot, bond_ref[...], preferred_element_type=jnp.float32)
    out_ref[...] = h * battr


def _edge_attr(d2, etype, W_d1, b_d1, W_d2, b_d2, bond_emb):
    E = d2.shape[0]
    H = W_d1.shape[1]
    nb = E // _BEA
    grid = (nb,)
    full = lambda i: (0, 0)
    return pl.pallas_call(
        _edge_attr_body,
        grid=grid,
        in_specs=[
            pl.BlockSpec((1, 1, _BEA), lambda i: (i, 0, 0)),
            pl.BlockSpec((1, 1, _BEA), lambda i: (i, 0, 0)),
            pl.BlockSpec((1, H), full),
            pl.BlockSpec((1, H), full),
            pl.BlockSpec((H, H), full),
            pl.BlockSpec((1, H), full),
            pl.BlockSpec((NB, H), full),
        ],
        out_specs=pl.BlockSpec((_BEA, H), lambda i: (i, 0)),
        out_shape=jax.ShapeDtypeStruct((E, H), jnp.float32),
    )(d2.reshape(nb, 1, _BEA), etype.reshape(nb, 1, _BEA),
      W_d1, b_d1.reshape(1, H), W_d2, b_d2.reshape(1, H), bond_emb)


def _node_body(z_ref, agg_ref, w_ref, b_ref, out_ref):
    x = z_ref[...] + agg_ref[0] + agg_ref[1]
    out_ref[...] = jax.nn.relu(
        jnp.dot(x, w_ref[...], preferred_element_type=jnp.float32) + b_ref[...])


def _node_update(z, agg2, W, b):
    Np, H = z.shape
    grid = (Np // _BN,)
    full = lambda i: (0, 0)
    return pl.pallas_call(
        _node_body,
        grid=grid,
        in_specs=[
            pl.BlockSpec((_BN, H), lambda i: (i, 0)),
            pl.BlockSpec((2, _BN, H), lambda i: (0, i, 0)),
            pl.BlockSpec((H, H), full),
            pl.BlockSpec((1, H), full),
        ],
        out_specs=pl.BlockSpec((_BN, H), lambda i: (i, 0)),
        out_shape=jax.ShapeDtypeStruct((Np, H), jnp.float32),
    )(z, agg2, W, b.reshape(1, H))


def _score_body(pair_ref, e2_ref, wa_ref, wb_ref, b1_ref, w2_ref, b2_ref,
                w3_ref, b3_ref, out_ref):
    x = jax.nn.relu(
        jnp.dot(pair_ref[...], wa_ref[...], preferred_element_type=jnp.float32)
        + jnp.dot(e2_ref[...], wb_ref[...], preferred_element_type=jnp.float32)
        + b1_ref[...])
    x = jax.nn.relu(
        jnp.dot(x, w2_ref[...], preferred_element_type=jnp.float32) + b2_ref[...])
    out_ref[...] = jnp.sum(x * w3_ref[...], axis=1, keepdims=True) + b3_ref[...]


def _score_mlp(pair, e2, W_s1, b_s1, W_s2, b_s2, W_s3, b_s3):
    E, H = pair.shape
    Hh = W_s2.shape[1]
    grid = (E // _BE,)
    full = lambda i: (0, 0)
    return pl.pallas_call(
        _score_body,
        grid=grid,
        in_specs=[
            pl.BlockSpec((_BE, H), lambda i: (i, 0)),
            pl.BlockSpec((_BE, H), lambda i: (i, 0)),
            pl.BlockSpec((H, H), full),
            pl.BlockSpec((H, H), full),
            pl.BlockSpec((1, H), full),
            pl.BlockSpec((H, Hh), full),
            pl.BlockSpec((1, Hh), full),
            pl.BlockSpec((1, Hh), full),
            pl.BlockSpec((1, 1), full),
        ],
        out_specs=pl.BlockSpec((_BE, 1), lambda i: (i, 0)),
        out_shape=jax.ShapeDtypeStruct((E, 1), jnp.float32),
    )(pair, e2, W_s1[:H], W_s1[H:], b_s1.reshape(1, H), W_s2,
      b_s2.reshape(1, Hh), W_s3.reshape(1, Hh), b_s3.reshape(1, 1))


def kernel(atom_type, pos, current_edge_index, current_edge_feat,
           full_edge_index, full_edge_type,
           atom_emb, bond_emb,
           W_d1, b_d1, W_d2, b_d2,
           W_e1, b_e1, W_e2, b_e2,
           W_s1, b_s1, W_s2, b_s2, W_s3, b_s3):
    n = atom_type.shape[0]
    e = current_edge_index.shape[1]

    z = atom_emb[atom_type]

    prows = (3 * n + 127) // 128
    pos_pad = jnp.zeros((prows * 128,), jnp.float32).at[:3 * n].set(
        pos.reshape(-1)).reshape(prows, 128)
    d2c, d2f = _make_edge_d2(n, e)(
        pos_pad, current_edge_index[0], current_edge_index[1],
        full_edge_index[0], full_edge_index[1])
    edge_attr = _edge_attr(d2c, current_edge_feat.astype(jnp.int32),
                           W_d1, b_d1, W_d2, b_d2, bond_emb)
    edge2 = _edge_attr(d2f, full_edge_type.astype(jnp.int32),
                       W_d1, b_d1, W_d2, b_d2, bond_emb)

    nchunks = e // (_NW * _CH)
    fsrc3 = full_edge_index[0].reshape(_NW, nchunks, _CH)
    fdst3 = full_edge_index[1].reshape(_NW, nchunks, _CH)
    n_pad = (n + 8 * _NS - 1) // (8 * _NS) * (8 * _NS)
    zeros = jnp.zeros((n_pad // _NS, z.shape[1]), jnp.float32)

    msg_agg = _make_msg_agg(n, e, z.shape[1])
    agg2 = msg_agg(z, edge_attr, current_edge_index[0],
                   current_edge_index[1], zeros)
    h = _node_update(z, agg2, W_e1, b_e1)

    agg2 = msg_agg(h, edge_attr, current_edge_index[0],
                   current_edge_index[1], zeros)
    node = _node_update(h, agg2, W_e2, b_e2)

    pair = _make_pair_prod(n, e, z.shape[1])(node, fsrc3, fdst3)
    return _score_mlp(pair, edge2, W_s1, b_s1, W_s2, b_s2, W_s3, b_s3)


# speedup vs baseline: 6.9007x; 1.0256x over previous
import functools

import jax
import jax.numpy as jnp
from jax import lax
from jax.experimental import pallas as pl
from jax.experimental.pallas import tpu as pltpu
from jax.experimental.pallas import tpu_sc as plsc

NB = 8

_BE = 4000
_BN = 2000

_NC = 2
_NS = 16
_NW = _NC * _NS
_CH = 80


_K = 4


def _mul_rows(dst_ref, a_ref, b_ref, nrows, h):
    @plsc.parallel_loop(0, nrows, 1, unroll=4)
    def mrow(r):
        for k in range(h // 16):
            sl = pl.ds(k * 16, 16)
            dst_ref[r, sl] = a_ref[r, sl] * b_ref[r, sl]


def _make_msg_agg(n, e, h):
    epw = e // _NW
    nchunks = epw // _CH
    ngroups = (nchunks + 3) // 4
    n_pad = (n + 8 * _NS - 1) // (8 * _NS) * (8 * _NS)
    rps = n_pad // _NS
    mesh = plsc.VectorSubcoreMesh(core_axis_name="c", subcore_axis_name="s")

    @functools.partial(
        pl.kernel, mesh=mesh,
        out_type=jax.ShapeDtypeStruct((_NC, n_pad, h), jnp.float32),
        scratch_types=(
            [pltpu.VMEM((_CH,), jnp.int32) for _ in range(8)]
            + [pltpu.VMEM((_CH, h), jnp.float32) for _ in range(4)]
            + [pltpu.VMEM_SHARED((n_pad, h), jnp.float32)]
            + [pltpu.SemaphoreType.DMA for _ in range(10)]
        ),
    )
    def msg_agg(z_hbm, ea_hbm, src_hbm, dst_hbm, zero_hbm, out_hbm, *bufs):
        sidx = bufs[0:4]
        didx = bufs[4:8]
        rows = bufs[8:10]
        eab = bufs[10:12]
        acc = bufs[12]
        sem_g = bufs[13:15]
        sem_e = bufs[15:17]
        sem_s = bufs[17:19]
        sem_i = bufs[19:23]
        c = lax.axis_index("c")
        s = lax.axis_index("s")
        wid = s * _NC + c
        base = wid * epw
        pltpu.sync_copy(zero_hbm, acc.at[pl.ds(s * rps, rps)])
        plsc.subcore_barrier()

        def issue_idx(i, q):
            off = base + i * _CH
            pltpu.async_copy(src_hbm.at[pl.ds(off, _CH)], sidx[q], sem_i[q])
            pltpu.async_copy(dst_hbm.at[pl.ds(off, _CH)], didx[q], sem_i[q])

        def wait_idx(q):
            pltpu.make_async_copy(
                src_hbm.at[pl.ds(0, _CH)], sidx[q], sem_i[q]).wait()
            pltpu.make_async_copy(
                dst_hbm.at[pl.ds(0, _CH)], didx[q], sem_i[q]).wait()

        def issue_gather(i, b, q):
            pltpu.async_copy(z_hbm.at[sidx[q]], rows[b], sem_g[b])

        def issue_ea(i, b):
            pltpu.async_copy(
                ea_hbm.at[pl.ds(base + i * _CH, _CH), :], eab[b], sem_e[b])

        issue_idx(0, 0)
        issue_idx(1, 1)
        issue_idx(2, 2)
        wait_idx(0)
        issue_gather(0, 0, 0)
        issue_ea(0, 0)
        wait_idx(1)
        issue_gather(1, 1, 1)

        def group(g, carry):
            for bb in range(4):
                i = g * 4 + bb
                b = bb % 2
                q = bb
                b1 = (bb + 1) % 2
                q2 = (bb + 2) % 4
                q3 = (bb + 3) % 4

                @pl.when(i < nchunks)
                def _main():
                    pltpu.make_async_copy(
                        z_hbm.at[pl.ds(0, _CH), :], rows[b], sem_g[b]).wait()
                    pltpu.make_async_copy(
                        ea_hbm.at[pl.ds(0, _CH), :], eab[b], sem_e[b]).wait()
                    _mul_rows(eab[b], rows[b], eab[b], _CH, h)
                    pltpu.async_copy(eab[b], acc.at[didx[q]], sem_s[b],
                                     add=True)

                    @pl.when(i >= 1)
                    def _drain_prev_scatter():
                        pltpu.make_async_copy(
                            z_hbm.at[pl.ds(0, _CH), :], eab[b1],
                            sem_s[b1]).wait()

                    @pl.when(i + 1 < nchunks)
                    def _prefetch_ea():
                        issue_ea(i + 1, b1)

                    @pl.when(i + 3 < nchunks)
                    def _prefetch_idx():
                        issue_idx(i + 3, q3)

                    @pl.when(i + 2 < nchunks)
                    def _prefetch_gather():
                        wait_idx(q2)
                        issue_gather(i + 2, b, q2)

            return carry

        lax.fori_loop(0, ngroups, group, 0)
        pltpu.make_async_copy(
            z_hbm.at[pl.ds(0, _CH), :], eab[(nchunks - 1) % 2],
            sem_s[(nchunks - 1) % 2]).wait()
        plsc.subcore_barrier()
        sl = pl.ds(s * rps, rps)
        pltpu.sync_copy(acc.at[sl], out_hbm.at[c, sl, :])

    return msg_agg


def _make_pair_prod(n, e, h):
    epw = e // _NW
    nchunks = epw // _CH
    ngroups = (nchunks + 3) // 4
    stage = n // 10
    mesh = plsc.VectorSubcoreMesh(core_axis_name="c", subcore_axis_name="s")

    @functools.partial(
        pl.kernel, mesh=mesh,
        out_type=jax.ShapeDtypeStruct((e, h), jnp.float32),
        scratch_types=(
            [pltpu.VMEM((_CH,), jnp.int32) for _ in range(8)]
            + [pltpu.VMEM((_CH, h), jnp.float32) for _ in range(4)]
            + [pltpu.VMEM_SHARED((n, h), jnp.float32)]
            + [pltpu.SemaphoreType.DMA for _ in range(10)]
        ),
    )
    def pair_prod(node_hbm, src_hbm, dst_hbm, out_hbm, *bufs):
        sidx = bufs[0:4]
        didx = bufs[4:8]
        rows = bufs[8:10]
        rows2 = bufs[10:12]
        nodes = bufs[12]
        sem_g = bufs[13:15]
        sem_g2 = bufs[15:17]
        sem_w = bufs[17:19]
        sem_i = bufs[19:23]
        c = lax.axis_index("c")
        s = lax.axis_index("s")
        wid = s * _NC + c
        base = wid * epw

        @pl.when(s < 10)
        def _stage():
            sl = pl.ds(s * stage, stage)
            pltpu.sync_copy(node_hbm.at[sl, :], nodes.at[sl])

        def issue_idx(i, q):
            off = base + i * _CH
            pltpu.async_copy(src_hbm.at[pl.ds(off, _CH)], sidx[q], sem_i[q])
            pltpu.async_copy(dst_hbm.at[pl.ds(off, _CH)], didx[q], sem_i[q])

        def wait_idx(q):
            pltpu.make_async_copy(
                src_hbm.at[pl.ds(0, _CH)], sidx[q], sem_i[q]).wait()
            pltpu.make_async_copy(
                dst_hbm.at[pl.ds(0, _CH)], didx[q], sem_i[q]).wait()

        def issue_ga(i, b, q):
            pltpu.async_copy(nodes.at[sidx[q]], rows[b], sem_g[b])

        def issue_gb(i, b, q):
            pltpu.async_copy(nodes.at[didx[q]], rows2[b], sem_g2[b])

        issue_idx(0, 0)
        issue_idx(1, 1)
        issue_idx(2, 2)
        plsc.subcore_barrier()
        wait_idx(0)
        issue_ga(0, 0, 0)
        issue_gb(0, 0, 0)
        wait_idx(1)
        issue_ga(1, 1, 1)

        def group(g, carry):
            for bb in range(4):
                i = g * 4 + bb
                b = bb % 2
                q = bb
                b1 = (bb + 1) % 2
                q1 = (bb + 1) % 4
                q2 = (bb + 2) % 4
                q3 = (bb + 3) % 4

                @pl.when(i < nchunks)
                def _main():
                    pltpu.make_async_copy(
                        node_hbm.at[pl.ds(0, _CH), :], rows[b],
                        sem_g[b]).wait()
                    pltpu.make_async_copy(
                        node_hbm.at[pl.ds(0, _CH), :], rows2[b],
                        sem_g2[b]).wait()
                    _mul_rows(rows2[b], rows[b], rows2[b], _CH, h)
                    pltpu.async_copy(
                        rows2[b], out_hbm.at[pl.ds(base + i * _CH, _CH), :],
                        sem_w[b])

                    @pl.when(i >= 1)
                    def _drain_prev_write():
                        pltpu.make_async_copy(
                            node_hbm.at[pl.ds(0, _CH), :], rows2[b1],
                            sem_w[b1]).wait()

                    @pl.when(i + 1 < nchunks)
                    def _prefetch_gb():
                        issue_gb(i + 1, b1, q1)

                    @pl.when(i + 3 < nchunks)
                    def _prefetch_idx():
                        issue_idx(i + 3, q3)

                    @pl.when(i + 2 < nchunks)
                    def _prefetch_ga():
                        wait_idx(q2)
                        issue_ga(i + 2, b, q2)

            return carry

        lax.fori_loop(0, ngroups, group, 0)
        pltpu.make_async_copy(
            node_hbm.at[pl.ds(0, _CH), :], rows2[(nchunks - 1) % 2],
            sem_w[(nchunks - 1) % 2]).wait()

    return pair_prod


def _make_edge_d2(n, e):
    epw = e // _NW
    nchunks = epw // _CH
    prows = (3 * n + 127) // 128
    mesh = plsc.VectorSubcoreMesh(core_axis_name="c", subcore_axis_name="s")

    @functools.partial(
        pl.kernel, mesh=mesh,
        compiler_params=pltpu.CompilerParams(needs_layout_passes=False),
        out_type=(jax.ShapeDtypeStruct((e,), jnp.float32),
                  jax.ShapeDtypeStruct((e,), jnp.float32)),
        scratch_types=[
            pltpu.VMEM((prows, 128), jnp.float32),
            pltpu.VMEM((epw,), jnp.int32),
            pltpu.VMEM((epw,), jnp.int32),
            pltpu.VMEM((epw,), jnp.float32),
        ],
    )
    def edge_d2(pos_hbm, srcc_hbm, dstc_hbm, srcf_hbm, dstf_hbm,
                outc_hbm, outf_hbm, posv, sidx, didx, d2v):
        c = lax.axis_index("c")
        s = lax.axis_index("s")
        wid = s * _NC + c
        pltpu.sync_copy(pos_hbm, posv)
        base = wid * epw

        for src_hbm, dst_hbm, out_hbm in ((srcc_hbm, dstc_hbm, outc_hbm),
                                          (srcf_hbm, dstf_hbm, outf_hbm)):
            pltpu.sync_copy(src_hbm.at[pl.ds(base, epw)], sidx)
            pltpu.sync_copy(dst_hbm.at[pl.ds(base, epw)], didx)

            def veci(k, carry2):
                sl = pl.ds(k * 16, 16)
                si = sidx[sl] * 3
                di = didx[sl] * 3
                acc = jnp.zeros((16,), jnp.float32)
                for j in range(3):
                    sij = si + j
                    dij = di + j
                    dp = (plsc.load_gather(posv, [sij >> 7, sij & 127])
                          - plsc.load_gather(posv, [dij >> 7, dij & 127]))
                    acc = acc + dp * dp
                d2v[sl] = acc
                return carry2

            lax.fori_loop(0, epw // 16, veci, 0)
            pltpu.sync_copy(d2v, out_hbm.at[pl.ds(base, epw)])

    return edge_d2


_BEA = 2560


def _edge_attr_body(d2_ref, et_ref, wd1_ref, bd1_ref, wd2_ref, bd2_ref,
                    bond_ref, out_ref):
    d = jnp.sqrt(d2_ref[...]).reshape(_BEA, 1)
    h = jax.nn.relu(d * wd1_ref[...] + bd1_ref[...])
    h = jnp.dot(h, wd2_ref[...], preferred_element_type=jnp.float32) + bd2_ref[...]
    et = et_ref[...].reshape(_BEA, 1)
    onehot = (et == jax.lax.broadcasted_iota(jnp.int32, (1, NB), 1)
              ).astype(jnp.float32)
    battr = jnp.dot(onehot, bond_ref[...], preferred_element_type=jnp.float32)
    out_ref[...] = h * battr


def _edge_attr(d2, etype, W_d1, b_d1, W_d2, b_d2, bond_emb):
    E = d2.shape[0]
    H = W_d1.shape[1]
    nb = E // _BEA
    grid = (nb,)
    full = lambda i: (0, 0)
    return pl.pallas_call(
        _edge_attr_body,
        grid=grid,
        in_specs=[
            pl.BlockSpec((1, 1, _BEA), lambda i: (i, 0, 0)),
            pl.BlockSpec((1, 1, _BEA), lambda i: (i, 0, 0)),
            pl.BlockSpec((1, H), full),
            pl.BlockSpec((1, H), full),
            pl.BlockSpec((H, H), full),
            pl.BlockSpec((1, H), full),
            pl.BlockSpec((NB, H), full),
        ],
        out_specs=pl.BlockSpec((_BEA, H), lambda i: (i, 0)),
        out_shape=jax.ShapeDtypeStruct((E, H), jnp.float32),
    )(d2.reshape(nb, 1, _BEA), etype.reshape(nb, 1, _BEA),
      W_d1, b_d1.reshape(1, H), W_d2, b_d2.reshape(1, H), bond_emb)


def _node_body(z_ref, agg_ref, w_ref, b_ref, out_ref):
    x = z_ref[...] + agg_ref[0] + agg_ref[1]
    out_ref[...] = jax.nn.relu(
        jnp.dot(x, w_ref[...], preferred_element_type=jnp.float32) + b_ref[...])


def _node_update(z, agg2, W, b):
    Np, H = z.shape
    grid = (Np // _BN,)
    full = lambda i: (0, 0)
    return pl.pallas_call(
        _node_body,
        grid=grid,
        in_specs=[
            pl.BlockSpec((_BN, H), lambda i: (i, 0)),
            pl.BlockSpec((2, _BN, H), lambda i: (0, i, 0)),
            pl.BlockSpec((H, H), full),
            pl.BlockSpec((1, H), full),
        ],
        out_specs=pl.BlockSpec((_BN, H), lambda i: (i, 0)),
        out_shape=jax.ShapeDtypeStruct((Np, H), jnp.float32),
    )(z, agg2, W, b.reshape(1, H))


def _score_body(pair_ref, e2_ref, wa_ref, wb_ref, b1_ref, w2_ref, b2_ref,
                w3_ref, b3_ref, out_ref):
    x = jax.nn.relu(
        jnp.dot(pair_ref[...], wa_ref[...], preferred_element_type=jnp.float32)
        + jnp.dot(e2_ref[...], wb_ref[...], preferred_element_type=jnp.float32)
        + b1_ref[...])
    x = jax.nn.relu(
        jnp.dot(x, w2_ref[...], preferred_element_type=jnp.float32) + b2_ref[...])
    out_ref[...] = jnp.sum(x * w3_ref[...], axis=1, keepdims=True) + b3_ref[...]


def _score_mlp(pair, e2, W_s1, b_s1, W_s2, b_s2, W_s3, b_s3):
    E, H = pair.shape
    Hh = W_s2.shape[1]
    grid = (E // _BE,)
    full = lambda i: (0, 0)
    return pl.pallas_call(
        _score_body,
        grid=grid,
        in_specs=[
            pl.BlockSpec((_BE, H), lambda i: (i, 0)),
            pl.BlockSpec((_BE, H), lambda i: (i, 0)),
            pl.BlockSpec((H, H), full),
            pl.BlockSpec((H, H), full),
            pl.BlockSpec((1, H), full),
            pl.BlockSpec((H, Hh), full),
            pl.BlockSpec((1, Hh), full),
            pl.BlockSpec((1, Hh), full),
            pl.BlockSpec((1, 1), full),
        ],
        out_specs=pl.BlockSpec((_BE, 1), lambda i: (i, 0)),
        out_shape=jax.ShapeDtypeStruct((E, 1), jnp.float32),
    )(pair, e2, W_s1[:H], W_s1[H:], b_s1.reshape(1, H), W_s2,
      b_s2.reshape(1, Hh), W_s3.reshape(1, Hh), b_s3.reshape(1, 1))


def kernel(atom_type, pos, current_edge_index, current_edge_feat,
           full_edge_index, full_edge_type,
           atom_emb, bond_emb,
           W_d1, b_d1, W_d2, b_d2,
           W_e1, b_e1, W_e2, b_e2,
           W_s1, b_s1, W_s2, b_s2, W_s3, b_s3):
    n = atom_type.shape[0]
    e = current_edge_index.shape[1]

    z = atom_emb[atom_type]

    prows = (3 * n + 127) // 128
    pos_pad = jnp.zeros((prows * 128,), jnp.float32).at[:3 * n].set(
        pos.reshape(-1)).reshape(prows, 128)
    d2c, d2f = _make_edge_d2(n, e)(
        pos_pad, current_edge_index[0], current_edge_index[1],
        full_edge_index[0], full_edge_index[1])
    edge_attr = _edge_attr(d2c, current_edge_feat.astype(jnp.int32),
                           W_d1, b_d1, W_d2, b_d2, bond_emb)
    edge2 = _edge_attr(d2f, full_edge_type.astype(jnp.int32),
                       W_d1, b_d1, W_d2, b_d2, bond_emb)

    n_pad = (n + 8 * _NS - 1) // (8 * _NS) * (8 * _NS)
    zeros = jnp.zeros((n_pad // _NS, z.shape[1]), jnp.float32)

    msg_agg = _make_msg_agg(n, e, z.shape[1])
    agg2 = msg_agg(z, edge_attr, current_edge_index[0],
                   current_edge_index[1], zeros)
    h = _node_update(z, agg2, W_e1, b_e1)

    agg2 = msg_agg(h, edge_attr, current_edge_index[0],
                   current_edge_index[1], zeros)
    node = _node_update(h, agg2, W_e2, b_e2)

    pair = _make_pair_prod(n, e, z.shape[1])(
        node, full_edge_index[0], full_edge_index[1])
    return _score_mlp(pair, edge2, W_s1, b_s1, W_s2, b_s2, W_s3, b_s3)


# speedup vs baseline: 6.9060x; 1.0008x over previous
import functools

import jax
import jax.numpy as jnp
from jax import lax
from jax.experimental import pallas as pl
from jax.experimental.pallas import tpu as pltpu
from jax.experimental.pallas import tpu_sc as plsc

NB = 8

_BE = 4000
_BN = 2000

_NC = 2
_NS = 16
_NW = _NC * _NS
_CH = 80


def _mul_rows(dst_ref, a_ref, b_ref, nrows, h):
    @plsc.parallel_loop(0, nrows, 1, unroll=4)
    def mrow(r):
        for k in range(h // 16):
            sl = pl.ds(k * 16, 16)
            dst_ref[r, sl] = a_ref[r, sl] * b_ref[r, sl]


def _make_msg_agg(n, e, h):
    epw = e // _NW
    nchunks = epw // _CH
    ngroups = (nchunks + 3) // 4
    n_pad = (n + 8 * _NS - 1) // (8 * _NS) * (8 * _NS)
    rps = n_pad // _NS
    mesh = plsc.VectorSubcoreMesh(core_axis_name="c", subcore_axis_name="s")

    @functools.partial(
        pl.kernel, mesh=mesh,
        out_type=jax.ShapeDtypeStruct((_NC, n_pad, h), jnp.float32),
        scratch_types=(
            [pltpu.VMEM((_CH,), jnp.int32) for _ in range(8)]
            + [pltpu.VMEM((_CH, h), jnp.float32) for _ in range(4)]
            + [pltpu.VMEM_SHARED((n_pad, h), jnp.float32)]
            + [pltpu.SemaphoreType.DMA for _ in range(10)]
        ),
    )
    def msg_agg(z_hbm, ea_hbm, src_hbm, dst_hbm, zero_hbm, out_hbm, *bufs):
        sidx = bufs[0:4]
        didx = bufs[4:8]
        rows = bufs[8:10]
        eab = bufs[10:12]
        acc = bufs[12]
        sem_g = bufs[13:15]
        sem_e = bufs[15:17]
        sem_s = bufs[17:19]
        sem_i = bufs[19:23]
        c = lax.axis_index("c")
        s = lax.axis_index("s")
        wid = s * _NC + c
        base = wid * epw
        pltpu.sync_copy(zero_hbm, acc.at[pl.ds(s * rps, rps)])
        plsc.subcore_barrier()

        def issue_idx(i, q):
            off = base + i * _CH
            pltpu.async_copy(src_hbm.at[pl.ds(off, _CH)], sidx[q], sem_i[q])
            pltpu.async_copy(dst_hbm.at[pl.ds(off, _CH)], didx[q], sem_i[q])

        def wait_idx(q):
            pltpu.make_async_copy(
                src_hbm.at[pl.ds(0, _CH)], sidx[q], sem_i[q]).wait()
            pltpu.make_async_copy(
                dst_hbm.at[pl.ds(0, _CH)], didx[q], sem_i[q]).wait()

        def issue_gather(i, b, q):
            pltpu.async_copy(z_hbm.at[sidx[q]], rows[b], sem_g[b])

        def issue_ea(i, b):
            pltpu.async_copy(
                ea_hbm.at[pl.ds(base + i * _CH, _CH), :], eab[b], sem_e[b])

        issue_idx(0, 0)
        issue_idx(1, 1)
        issue_idx(2, 2)
        wait_idx(0)
        issue_gather(0, 0, 0)
        issue_ea(0, 0)
        wait_idx(1)
        issue_gather(1, 1, 1)

        def group(g, carry):
            for bb in range(4):
                i = g * 4 + bb
                b = bb % 2
                q = bb
                b1 = (bb + 1) % 2
                q2 = (bb + 2) % 4
                q3 = (bb + 3) % 4

                @pl.when(i < nchunks)
                def _main():
                    pltpu.make_async_copy(
                        z_hbm.at[pl.ds(0, _CH), :], rows[b], sem_g[b]).wait()
                    pltpu.make_async_copy(
                        ea_hbm.at[pl.ds(0, _CH), :], eab[b], sem_e[b]).wait()
                    _mul_rows(eab[b], rows[b], eab[b], _CH, h)
                    pltpu.async_copy(eab[b], acc.at[didx[q]], sem_s[b],
                                     add=True)

                    @pl.when(i >= 1)
                    def _drain_prev_scatter():
                        pltpu.make_async_copy(
                            z_hbm.at[pl.ds(0, _CH), :], eab[b1],
                            sem_s[b1]).wait()

                    @pl.when(i + 1 < nchunks)
                    def _prefetch_ea():
                        issue_ea(i + 1, b1)

                    @pl.when(i + 3 < nchunks)
                    def _prefetch_idx():
                        issue_idx(i + 3, q3)

                    @pl.when(i + 2 < nchunks)
                    def _prefetch_gather():
                        wait_idx(q2)
                        issue_gather(i + 2, b, q2)

            return carry

        lax.fori_loop(0, ngroups, group, 0)
        pltpu.make_async_copy(
            z_hbm.at[pl.ds(0, _CH), :], eab[(nchunks - 1) % 2],
            sem_s[(nchunks - 1) % 2]).wait()
        plsc.subcore_barrier()
        sl = pl.ds(s * rps, rps)
        pltpu.sync_copy(acc.at[sl], out_hbm.at[c, sl, :])

    return msg_agg


def _make_pair_prod(n, e, h):
    epw = e // _NW
    nchunks = epw // _CH
    ngroups = (nchunks + 3) // 4
    stage = n // 10
    mesh = plsc.VectorSubcoreMesh(core_axis_name="c", subcore_axis_name="s")

    @functools.partial(
        pl.kernel, mesh=mesh,
        out_type=jax.ShapeDtypeStruct((e, h), jnp.float32),
        scratch_types=(
            [pltpu.VMEM((_CH,), jnp.int32) for _ in range(8)]
            + [pltpu.VMEM((_CH, h), jnp.float32) for _ in range(4)]
            + [pltpu.VMEM_SHARED((n, h), jnp.float32)]
            + [pltpu.SemaphoreType.DMA for _ in range(10)]
        ),
    )
    def pair_prod(node_hbm, src_hbm, dst_hbm, out_hbm, *bufs):
        sidx = bufs[0:4]
        didx = bufs[4:8]
        rows = bufs[8:10]
        rows2 = bufs[10:12]
        nodes = bufs[12]
        sem_g = bufs[13:15]
        sem_g2 = bufs[15:17]
        sem_w = bufs[17:19]
        sem_i = bufs[19:23]
        c = lax.axis_index("c")
        s = lax.axis_index("s")
        wid = s * _NC + c
        base = wid * epw

        @pl.when(s < 10)
        def _stage():
            sl = pl.ds(s * stage, stage)
            pltpu.sync_copy(node_hbm.at[sl, :], nodes.at[sl])

        def issue_idx(i, q):
            off = base + i * _CH
            pltpu.async_copy(src_hbm.at[pl.ds(off, _CH)], sidx[q], sem_i[q])
            pltpu.async_copy(dst_hbm.at[pl.ds(off, _CH)], didx[q], sem_i[q])

        def wait_idx(q):
            pltpu.make_async_copy(
                src_hbm.at[pl.ds(0, _CH)], sidx[q], sem_i[q]).wait()
            pltpu.make_async_copy(
                dst_hbm.at[pl.ds(0, _CH)], didx[q], sem_i[q]).wait()

        def issue_ga(i, b, q):
            pltpu.async_copy(nodes.at[sidx[q]], rows[b], sem_g[b])

        def issue_gb(i, b, q):
            pltpu.async_copy(nodes.at[didx[q]], rows2[b], sem_g2[b])

        issue_idx(0, 0)
        issue_idx(1, 1)
        issue_idx(2, 2)
        plsc.subcore_barrier()
        wait_idx(0)
        issue_ga(0, 0, 0)
        issue_gb(0, 0, 0)
        wait_idx(1)
        issue_ga(1, 1, 1)

        def group(g, carry):
            for bb in range(4):
                i = g * 4 + bb
                b = bb % 2
                q = bb
                b1 = (bb + 1) % 2
                q1 = (bb + 1) % 4
                q2 = (bb + 2) % 4
                q3 = (bb + 3) % 4

                @pl.when(i < nchunks)
                def _main():
                    pltpu.make_async_copy(
                        node_hbm.at[pl.ds(0, _CH), :], rows[b],
                        sem_g[b]).wait()
                    pltpu.make_async_copy(
                        node_hbm.at[pl.ds(0, _CH), :], rows2[b],
                        sem_g2[b]).wait()
                    _mul_rows(rows2[b], rows[b], rows2[b], _CH, h)
                    pltpu.async_copy(
                        rows2[b], out_hbm.at[pl.ds(base + i * _CH, _CH), :],
                        sem_w[b])

                    @pl.when(i >= 1)
                    def _drain_prev_write():
                        pltpu.make_async_copy(
                            node_hbm.at[pl.ds(0, _CH), :], rows2[b1],
                            sem_w[b1]).wait()

                    @pl.when(i + 1 < nchunks)
                    def _prefetch_gb():
                        issue_gb(i + 1, b1, q1)

                    @pl.when(i + 3 < nchunks)
                    def _prefetch_idx():
                        issue_idx(i + 3, q3)

                    @pl.when(i + 2 < nchunks)
                    def _prefetch_ga():
                        wait_idx(q2)
                        issue_ga(i + 2, b, q2)

            return carry

        lax.fori_loop(0, ngroups, group, 0)
        pltpu.make_async_copy(
            node_hbm.at[pl.ds(0, _CH), :], rows2[(nchunks - 1) % 2],
            sem_w[(nchunks - 1) % 2]).wait()

    return pair_prod


def _make_edge_d2(n, e):
    epw = e // _NW
    nchunks = epw // _CH
    prows = (3 * n + 127) // 128
    mesh = plsc.VectorSubcoreMesh(core_axis_name="c", subcore_axis_name="s")

    @functools.partial(
        pl.kernel, mesh=mesh,
        compiler_params=pltpu.CompilerParams(needs_layout_passes=False),
        out_type=(jax.ShapeDtypeStruct((e,), jnp.float32),
                  jax.ShapeDtypeStruct((e,), jnp.float32)),
        scratch_types=[
            pltpu.VMEM((prows, 128), jnp.float32),
            pltpu.VMEM((epw,), jnp.int32),
            pltpu.VMEM((epw,), jnp.int32),
            pltpu.VMEM((epw,), jnp.float32),
        ],
    )
    def edge_d2(pos_hbm, srcc_hbm, dstc_hbm, srcf_hbm, dstf_hbm,
                outc_hbm, outf_hbm, posv, sidx, didx, d2v):
        c = lax.axis_index("c")
        s = lax.axis_index("s")
        wid = s * _NC + c
        pltpu.sync_copy(pos_hbm, posv)
        base = wid * epw

        for src_hbm, dst_hbm, out_hbm in ((srcc_hbm, dstc_hbm, outc_hbm),
                                          (srcf_hbm, dstf_hbm, outf_hbm)):
            pltpu.sync_copy(src_hbm.at[pl.ds(base, epw)], sidx)
            pltpu.sync_copy(dst_hbm.at[pl.ds(base, epw)], didx)

            def veci(k, carry2):
                sl = pl.ds(k * 16, 16)
                si = sidx[sl] * 3
                di = didx[sl] * 3
                acc = jnp.zeros((16,), jnp.float32)
                for j in range(3):
                    sij = si + j
                    dij = di + j
                    dp = (plsc.load_gather(posv, [sij >> 7, sij & 127])
                          - plsc.load_gather(posv, [dij >> 7, dij & 127]))
                    acc = acc + dp * dp
                d2v[sl] = acc
                return carry2

            lax.fori_loop(0, epw // 16, veci, 0)
            pltpu.sync_copy(d2v, out_hbm.at[pl.ds(base, epw)])

    return edge_d2


_BEA = 2560


def _edge_attr_body(d2_ref, et_ref, wd1_ref, bd1_ref, wd2_ref, bd2_ref,
                    bond_ref, out_ref):
    d = jnp.sqrt(d2_ref[...]).reshape(_BEA, 1)
    h = jax.nn.relu(d * wd1_ref[...] + bd1_ref[...])
    h = jnp.dot(h, wd2_ref[...], preferred_element_type=jnp.float32) + bd2_ref[...]
    et = et_ref[...].reshape(_BEA, 1)
    onehot = (et == jax.lax.broadcasted_iota(jnp.int32, (1, NB), 1)
              ).astype(jnp.float32)
    battr = jnp.dot(onehot, bond_ref[...], preferred_element_type=jnp.float32)
    out_ref[...] = h * battr


def _edge_attr(d2, etype, W_d1, b_d1, W_d2, b_d2, bond_emb):
    E = d2.shape[0]
    H = W_d1.shape[1]
    nb = E // _BEA
    grid = (nb,)
    full = lambda i: (0, 0)
    return pl.pallas_call(
        _edge_attr_body,
        grid=grid,
        in_specs=[
            pl.BlockSpec((1, 1, _BEA), lambda i: (i, 0, 0)),
            pl.BlockSpec((1, 1, _BEA), lambda i: (i, 0, 0)),
            pl.BlockSpec((1, H), full),
            pl.BlockSpec((1, H), full),
            pl.BlockSpec((H, H), full),
            pl.BlockSpec((1, H), full),
            pl.BlockSpec((NB, H), full),
        ],
        out_specs=pl.BlockSpec((_BEA, H), lambda i: (i, 0)),
        out_shape=jax.ShapeDtypeStruct((E, H), jnp.float32),
    )(d2.reshape(nb, 1, _BEA), etype.reshape(nb, 1, _BEA),
      W_d1, b_d1.reshape(1, H), W_d2, b_d2.reshape(1, H), bond_emb)


def _node_body(z_ref, agg_ref, w_ref, b_ref, out_ref):
    x = z_ref[...] + agg_ref[0] + agg_ref[1]
    out_ref[...] = jax.nn.relu(
        jnp.dot(x, w_ref[...], preferred_element_type=jnp.float32) + b_ref[...])


def _node_update(z, agg2, W, b):
    Np, H = z.shape
    grid = (Np // _BN,)
    full = lambda i: (0, 0)
    return pl.pallas_call(
        _node_body,
        grid=grid,
        in_specs=[
            pl.BlockSpec((_BN, H), lambda i: (i, 0)),
            pl.BlockSpec((2, _BN, H), lambda i: (0, i, 0)),
            pl.BlockSpec((H, H), full),
            pl.BlockSpec((1, H), full),
        ],
        out_specs=pl.BlockSpec((_BN, H), lambda i: (i, 0)),
        out_shape=jax.ShapeDtypeStruct((Np, H), jnp.float32),
    )(z, agg2, W, b.reshape(1, H))


def _score_body(pair_ref, e2_ref, wa_ref, wb_ref, b1_ref, w2_ref, b2_ref,
                w3_ref, b3_ref, out_ref):
    x = jax.nn.relu(
        jnp.dot(pair_ref[...], wa_ref[...], preferred_element_type=jnp.float32)
        + jnp.dot(e2_ref[...], wb_ref[...], preferred_element_type=jnp.float32)
        + b1_ref[...])
    x = jax.nn.relu(
        jnp.dot(x, w2_ref[...], preferred_element_type=jnp.float32) + b2_ref[...])
    out_ref[...] = jnp.sum(x * w3_ref[...], axis=1, keepdims=True) + b3_ref[...]


def _score_mlp(pair, e2, W_s1, b_s1, W_s2, b_s2, W_s3, b_s3):
    E, H = pair.shape
    Hh = W_s2.shape[1]
    grid = (E // _BE,)
    full = lambda i: (0, 0)
    return pl.pallas_call(
        _score_body,
        grid=grid,
        in_specs=[
            pl.BlockSpec((_BE, H), lambda i: (i, 0)),
            pl.BlockSpec((_BE, H), lambda i: (i, 0)),
            pl.BlockSpec((H, H), full),
            pl.BlockSpec((H, H), full),
            pl.BlockSpec((1, H), full),
            pl.BlockSpec((H, Hh), full),
            pl.BlockSpec((1, Hh), full),
            pl.BlockSpec((1, Hh), full),
            pl.BlockSpec((1, 1), full),
        ],
        out_specs=pl.BlockSpec((_BE, 1), lambda i: (i, 0)),
        out_shape=jax.ShapeDtypeStruct((E, 1), jnp.float32),
    )(pair, e2, W_s1[:H], W_s1[H:], b_s1.reshape(1, H), W_s2,
      b_s2.reshape(1, Hh), W_s3.reshape(1, Hh), b_s3.reshape(1, 1))


def kernel(atom_type, pos, current_edge_index, current_edge_feat,
           full_edge_index, full_edge_type,
           atom_emb, bond_emb,
           W_d1, b_d1, W_d2, b_d2,
           W_e1, b_e1, W_e2, b_e2,
           W_s1, b_s1, W_s2, b_s2, W_s3, b_s3):
    n = atom_type.shape[0]
    e = current_edge_index.shape[1]

    z = atom_emb[atom_type]

    prows = (3 * n + 127) // 128
    pos_pad = jnp.zeros((prows * 128,), jnp.float32).at[:3 * n].set(
        pos.reshape(-1)).reshape(prows, 128)
    d2c, d2f = _make_edge_d2(n, e)(
        pos_pad, current_edge_index[0], current_edge_index[1],
        full_edge_index[0], full_edge_index[1])
    edge_attr = _edge_attr(d2c, current_edge_feat.astype(jnp.int32),
                           W_d1, b_d1, W_d2, b_d2, bond_emb)
    edge2 = _edge_attr(d2f, full_edge_type.astype(jnp.int32),
                       W_d1, b_d1, W_d2, b_d2, bond_emb)

    n_pad = (n + 8 * _NS - 1) // (8 * _NS) * (8 * _NS)
    zeros = jnp.zeros((n_pad // _NS, z.shape[1]), jnp.float32)

    msg_agg = _make_msg_agg(n, e, z.shape[1])
    agg2 = msg_agg(z, edge_attr, current_edge_index[0],
                   current_edge_index[1], zeros)
    h = _node_update(z, agg2, W_e1, b_e1)

    agg2 = msg_agg(h, edge_attr, current_edge_index[0],
                   current_edge_index[1], zeros)
    node = _node_update(h, agg2, W_e2, b_e2)

    pair = _make_pair_prod(n, e, z.shape[1])(
        node, full_edge_index[0], full_edge_index[1])
    return _score_mlp(pair, edge2, W_s1, b_s1, W_s2, b_s2, W_s3, b_s3)
